# baseline (speedup 1.0000x reference)
import jax
import jax.numpy as jnp
from jax.experimental import pallas as pl

N_K = 1024
D = 256
B = 16384
BT = 4096


def _vq_body(z_ref, emb_ref, zq_ref, idx_ref):
    z = z_ref[...]
    emb = emb_ref[...]
    zsq = jnp.sum(z * z, axis=-1, keepdims=True)
    esq = jnp.sum(emb * emb, axis=-1)
    zb2 = z.astype(jnp.bfloat16)
    zb2 = zb2 + zb2
    s2 = jax.lax.dot_general(
        zb2, emb.astype(jnp.bfloat16),
        (((1,), (1,)), ((), ())),
        preferred_element_type=jnp.float32)
    d = (zsq + esq[None, :]) - s2
    m = jnp.min(d, axis=-1, keepdims=True)
    iota_f = jax.lax.broadcasted_iota(
        jnp.int32, d.shape, 1).astype(jnp.float32)
    idx_f = jnp.min(jnp.where(d == m, iota_f, float(N_K)), axis=-1,
                    keepdims=True)
    idx_ref[...] = idx_f[:, 0].astype(jnp.int32)
    onehot = (iota_f == idx_f).astype(jnp.float32)
    zq_ref[...] = jax.lax.dot_general(
        onehot, emb, (((1,), (0,)), ((), ())),
        preferred_element_type=jnp.float32)


def kernel(z, emb):
    zq, idx = pl.pallas_call(
        _vq_body,
        grid=(B // BT,),
        in_specs=[
            pl.BlockSpec((BT, D), lambda i: (i, 0)),
            pl.BlockSpec((N_K, D), lambda i: (0, 0)),
        ],
        out_specs=[
            pl.BlockSpec((BT, D), lambda i: (i, 0)),
            pl.BlockSpec((BT,), lambda i: (i,)),
        ],
        out_shape=[
            jax.ShapeDtypeStruct((B, D), jnp.float32),
            jax.ShapeDtypeStruct((B,), jnp.int32),
        ],
    )(z, emb)
    return (zq, idx)
